# baseline (speedup 1.0000x reference)
import jax
import jax.numpy as jnp
from jax.experimental import pallas as pl


B, N, D_IN, H, D_OUT = 64, 256, 128, 256, 128


def _fused_kernel(A_ref, x_ref, W1_ref, b1_ref, W2_ref, b2_ref,
                  W3_ref, b3_ref, Wout_ref, bout_ref, out_ref):
    A = A_ref[0]
    h = x_ref[0]

    agg = jnp.dot(A, h, preferred_element_type=jnp.float32) + h
    h = jax.nn.relu(
        jnp.dot(agg, W1_ref[...], preferred_element_type=jnp.float32)
        + b1_ref[...])
    agg = jnp.dot(A, h, preferred_element_type=jnp.float32) + h
    h = jax.nn.relu(
        jnp.dot(agg, W2_ref[...], preferred_element_type=jnp.float32)
        + b2_ref[...])
    agg = jnp.dot(A, h, preferred_element_type=jnp.float32) + h
    h = jax.nn.relu(
        jnp.dot(agg, W3_ref[...], preferred_element_type=jnp.float32)
        + b3_ref[...])

    hg = jnp.sum(h, axis=0, keepdims=True)
    out_ref[0] = (
        jnp.dot(hg, Wout_ref[...], preferred_element_type=jnp.float32)
        + bout_ref[...])


def kernel(G, x, W1, b1, W2, b2, W3, b3, Wout, bout):
    A = jnp.squeeze(G, axis=-1)
    b1r = b1.reshape(1, H)
    b2r = b2.reshape(1, H)
    b3r = b3.reshape(1, H)
    boutr = bout.reshape(1, D_OUT)

    const = lambda shape: pl.BlockSpec(shape, lambda i: (0,) * len(shape))
    out = pl.pallas_call(
        _fused_kernel,
        grid=(B,),
        in_specs=[
            pl.BlockSpec((1, N, N), lambda i: (i, 0, 0)),
            pl.BlockSpec((1, N, D_IN), lambda i: (i, 0, 0)),
            const((D_IN, H)), const((1, H)),
            const((H, H)), const((1, H)),
            const((H, H)), const((1, H)),
            const((H, D_OUT)), const((1, D_OUT)),
        ],
        out_specs=pl.BlockSpec((1, 1, D_OUT), lambda i: (i, 0, 0)),
        out_shape=jax.ShapeDtypeStruct((B, 1, D_OUT), jnp.float32),
    )(A, x, W1, b1r, W2, b2r, W3, b3r, Wout, boutr)
    out = out.reshape(B, D_OUT)

    side_loss = jnp.asarray(0.0, dtype=jnp.float32)
    return (out, side_loss)


# speedup vs baseline: 1.9467x; 1.9467x over previous
import jax
import jax.numpy as jnp
from jax.experimental import pallas as pl


B, N, D_IN, H, D_OUT = 64, 256, 128, 256, 128
BB = 8

_BATCHED = (((2,), (1,)), ((0,), (0,)))


def _fused_kernel(A_ref, x_ref, W1_ref, b1_ref, W2_ref, b2_ref,
                  W3_ref, b3_ref, Wout_ref, bout_ref, out_ref):
    A = A_ref[...]
    h = x_ref[...]

    def gin_layer(h, W_ref, b_ref):
        d = h.shape[-1]
        agg = jax.lax.dot_general(
            A, h, _BATCHED, preferred_element_type=jnp.float32) + h
        hf = jnp.dot(agg.reshape(BB * N, d), W_ref[...],
                     preferred_element_type=jnp.float32) + b_ref[...]
        return jax.nn.relu(hf).reshape(BB, N, H)

    h = gin_layer(h, W1_ref, b1_ref)
    h = gin_layer(h, W2_ref, b2_ref)
    h = gin_layer(h, W3_ref, b3_ref)

    hg = jnp.sum(h, axis=1)
    out_ref[...] = (
        jnp.dot(hg, Wout_ref[...], preferred_element_type=jnp.float32)
        + bout_ref[...])


def kernel(G, x, W1, b1, W2, b2, W3, b3, Wout, bout):
    A = jnp.squeeze(G, axis=-1)
    b1r = b1.reshape(1, H)
    b2r = b2.reshape(1, H)
    b3r = b3.reshape(1, H)
    boutr = bout.reshape(1, D_OUT)

    const = lambda shape: pl.BlockSpec(shape, lambda i: (0,) * len(shape))
    out = pl.pallas_call(
        _fused_kernel,
        grid=(B // BB,),
        in_specs=[
            pl.BlockSpec((BB, N, N), lambda i: (i, 0, 0)),
            pl.BlockSpec((BB, N, D_IN), lambda i: (i, 0, 0)),
            const((D_IN, H)), const((1, H)),
            const((H, H)), const((1, H)),
            const((H, H)), const((1, H)),
            const((H, D_OUT)), const((1, D_OUT)),
        ],
        out_specs=pl.BlockSpec((BB, D_OUT), lambda i: (i, 0)),
        out_shape=jax.ShapeDtypeStruct((B, D_OUT), jnp.float32),
    )(A, x, W1, b1r, W2, b2r, W3, b3r, Wout, boutr)

    side_loss = jnp.asarray(0.0, dtype=jnp.float32)
    return (out, side_loss)


# speedup vs baseline: 1.9898x; 1.0221x over previous
import jax
import jax.numpy as jnp
from jax.experimental import pallas as pl


B, N, D_IN, H, D_OUT = 64, 256, 128, 256, 128
BB = 16

_BATCHED = (((2,), (1,)), ((0,), (0,)))


def _fused_kernel(A_ref, x_ref, W1_ref, b1_ref, W2_ref, b2_ref,
                  W3_ref, b3_ref, Wout_ref, bout_ref, out_ref):
    A = A_ref[...]
    h = x_ref[...]

    def gin_layer(h, W_ref, b_ref):
        d = h.shape[-1]
        agg = jax.lax.dot_general(
            A, h, _BATCHED, preferred_element_type=jnp.float32) + h
        hf = jnp.dot(agg.reshape(BB * N, d), W_ref[...],
                     preferred_element_type=jnp.float32) + b_ref[...]
        return jax.nn.relu(hf).reshape(BB, N, H)

    h = gin_layer(h, W1_ref, b1_ref)
    h = gin_layer(h, W2_ref, b2_ref)
    h = gin_layer(h, W3_ref, b3_ref)

    hg = jnp.sum(h, axis=1)
    out_ref[...] = (
        jnp.dot(hg, Wout_ref[...], preferred_element_type=jnp.float32)
        + bout_ref[...])


def kernel(G, x, W1, b1, W2, b2, W3, b3, Wout, bout):
    A = jnp.squeeze(G, axis=-1)
    b1r = b1.reshape(1, H)
    b2r = b2.reshape(1, H)
    b3r = b3.reshape(1, H)
    boutr = bout.reshape(1, D_OUT)

    const = lambda shape: pl.BlockSpec(shape, lambda i: (0,) * len(shape))
    out = pl.pallas_call(
        _fused_kernel,
        grid=(B // BB,),
        in_specs=[
            pl.BlockSpec((BB, N, N), lambda i: (i, 0, 0)),
            pl.BlockSpec((BB, N, D_IN), lambda i: (i, 0, 0)),
            const((D_IN, H)), const((1, H)),
            const((H, H)), const((1, H)),
            const((H, H)), const((1, H)),
            const((H, D_OUT)), const((1, D_OUT)),
        ],
        out_specs=pl.BlockSpec((BB, D_OUT), lambda i: (i, 0)),
        out_shape=jax.ShapeDtypeStruct((B, D_OUT), jnp.float32),
    )(A, x, W1, b1r, W2, b2r, W3, b3r, Wout, boutr)

    side_loss = jnp.asarray(0.0, dtype=jnp.float32)
    return (out, side_loss)
